# baseline (speedup 1.0000x reference)
import functools

import jax
import jax.numpy as jnp
from jax import lax
from jax.experimental import pallas as pl
from jax.experimental.pallas import tpu as pltpu
from jax.experimental.pallas import tpu_sc as plsc

_NC = 2
_NS = 16
_NW = _NC * _NS
_CHUNK = 120
_NBUF = 3
_NSLICE = 8

_D_VAR = 128
_OUT = 256


def _sc_gather(table, idx_full, n_rows, slice_base):
    per_w = n_rows // _NW
    k_chunks = per_w // _CHUNK
    assert per_w % _CHUNK == 0 and k_chunks % _NBUF == 0
    mesh = plsc.VectorSubcoreMesh(core_axis_name="c", subcore_axis_name="s")

    @functools.partial(
        pl.kernel,
        mesh=mesh,
        out_type=jax.ShapeDtypeStruct((n_rows, _D_VAR), jnp.float32),
        scratch_types=[
            pltpu.VMEM((per_w,), jnp.int32),
        ] + [pltpu.VMEM((_CHUNK, _D_VAR), jnp.float32)] * _NBUF
          + [pltpu.SemaphoreType.DMA] * (2 * _NBUF),
    )
    def gather_kernel(table_hbm, idx_hbm, out_hbm, idxall, r0, r1, r2,
                      g0, g1, g2, w0, w1, w2):
        rows = [r0, r1, r2]
        gsem = [g0, g1, g2]
        wsem = [w0, w1, w2]
        wid = lax.axis_index("s") * _NC + lax.axis_index("c")
        base = wid * per_w

        pltpu.sync_copy(idx_hbm.at[pl.ds(slice_base + base, per_w)], idxall)

        def fire_gather(j, b):
            pltpu.async_copy(
                table_hbm.at[idxall.at[pl.ds(j * _CHUNK, _CHUNK)]],
                rows[b], gsem[b])

        def fire_wb(j, b):
            pltpu.async_copy(
                rows[b], out_hbm.at[pl.ds(base + j * _CHUNK, _CHUNK)],
                wsem[b])

        def wait_g(b):
            pltpu.make_async_copy(
                table_hbm.at[idxall.at[pl.ds(0, _CHUNK)]], rows[b],
                gsem[b]).wait()

        def wait_w(b):
            pltpu.make_async_copy(
                rows[b], out_hbm.at[pl.ds(base, _CHUNK)], wsem[b]).wait()

        def body(g, carry):
            for b in range(_NBUF):
                jb = _NBUF * g + b
                pb = (b + _NBUF - 1) % _NBUF

                @pl.when(g > 0)
                def _():
                    wait_w(b)

                fire_gather(jb, b)

                if b == 0:
                    @pl.when(g > 0)
                    def _():
                        wait_g(pb)
                        fire_wb(_NBUF * g - 1, pb)
                else:
                    wait_g(pb)
                    fire_wb(jb - 1, pb)
            return carry

        lax.fori_loop(0, k_chunks // _NBUF, body, 0)
        wait_g(_NBUF - 1)
        fire_wb(k_chunks - 1, _NBUF - 1)
        for b in range(_NBUF):
            wait_w(b)

    return gather_kernel(table, idx_full)


def _dot_t(lhs, rhs):
    return lax.dot_general(lhs, rhs, (((0,), (0,)), ((), ())),
                           preferred_element_type=jnp.float32)


def _tc_compute(g_ref, c1_ref, c2_ref, vaf_ref, vc_ref, f_ref, w_ref, b_ref,
                o_ref):
    L = o_ref.shape[1]
    g = g_ref[0, 0:L, :]
    acc = jnp.dot(g, w_ref[0:128, :], preferred_element_type=jnp.float32)

    c1 = c1_ref[0]
    c2 = c2_ref[0]

    vc_iota = lax.broadcasted_iota(jnp.int32, (33, L), 0)
    oh_vc = ((c1 & 63) == vc_iota).astype(jnp.float32)
    wvc = jnp.dot(vc_ref[...], w_ref[128:160, :],
                  preferred_element_type=jnp.float32)
    acc += _dot_t(oh_vc, wvc)

    f_iota = lax.broadcasted_iota(jnp.int32, (65, L), 0)
    counts = (((c1 >> 6) & 127) == f_iota).astype(jnp.float32)
    counts += (((c1 >> 13) & 127) == f_iota).astype(jnp.float32)
    counts += (((c1 >> 20) & 127) == f_iota).astype(jnp.float32)
    counts += ((c2 & 127) == f_iota).astype(jnp.float32)
    counts += (((c2 >> 7) & 127) == f_iota).astype(jnp.float32)
    counts += (((c2 >> 14) & 127) == f_iota).astype(jnp.float32)
    wf = jnp.dot(f_ref[...], w_ref[160:192, :],
                 preferred_element_type=jnp.float32) * (1.0 / 6.0)
    acc += _dot_t(counts, wf)

    acc += _dot_t(vaf_ref[0], w_ref[192:193, :])
    acc += b_ref[...]

    o_ref[0] = jnp.where(acc > 0.0, acc, jnp.exp(acc) - 1.0)


def _tc_body_first(g_ref, c1_ref, c2_ref, vaf_ref, vc_ref, f_ref, w_ref,
                   b_ref, o_ref):
    _tc_compute(g_ref, c1_ref, c2_ref, vaf_ref, vc_ref, f_ref, w_ref, b_ref,
                o_ref)


def _tc_body_acc(o_prev_ref, g_ref, c1_ref, c2_ref, vaf_ref, vc_ref, f_ref,
                 w_ref, b_ref, o_ref):
    del o_prev_ref
    _tc_compute(g_ref, c1_ref, c2_ref, vaf_ref, vc_ref, f_ref, w_ref, b_ref,
                o_ref)


def kernel(x_omic, emb_var, emb_vc, emb_func, W, b):
    B, L, _ = x_omic.shape
    step = 1440 if L <= 1440 else -(-L // 1440) * 1440
    l_pad = step
    n_pad = B * l_pad
    bs = B // _NSLICE
    n_sl = bs * l_pad

    var_ids = x_omic[..., 0].astype(jnp.int32)
    var_ids = jnp.pad(var_ids, ((0, 0), (0, l_pad - L)))
    idx_flat = var_ids.reshape(n_pad)

    ids = x_omic[..., 1:8].astype(jnp.int32)
    c1 = (ids[..., 0] | (ids[..., 1] << 6) | (ids[..., 2] << 13)
          | (ids[..., 3] << 20)).reshape(B, 1, L)
    c2 = (ids[..., 4] | (ids[..., 5] << 7)
          | (ids[..., 6] << 14)).reshape(B, 1, L)
    vaf3 = x_omic[..., 8].reshape(B, 1, L)

    b2 = b.reshape(1, _OUT)
    out_shape = jax.ShapeDtypeStruct((B, L, _OUT), jnp.float32)

    def specs(off):
        return [
            pl.BlockSpec((1, l_pad, _D_VAR), lambda i: (i, 0, 0)),
            pl.BlockSpec((1, 1, L), lambda i, o=off: (i + o, 0, 0)),
            pl.BlockSpec((1, 1, L), lambda i, o=off: (i + o, 0, 0)),
            pl.BlockSpec((1, 1, L), lambda i, o=off: (i + o, 0, 0)),
            pl.BlockSpec((33, 32), lambda i: (0, 0)),
            pl.BlockSpec((65, 32), lambda i: (0, 0)),
            pl.BlockSpec((193, _OUT), lambda i: (0, 0)),
            pl.BlockSpec((1, _OUT), lambda i: (0, 0)),
        ]

    out = None
    for s in range(_NSLICE):
        gath = _sc_gather(emb_var, idx_flat, n_sl, s * n_sl)
        g3 = gath.reshape(bs, l_pad, _D_VAR)
        off = s * bs
        out_spec = pl.BlockSpec((1, L, _OUT), lambda i, o=off: (i + o, 0, 0))
        if s == 0:
            out = pl.pallas_call(
                _tc_body_first,
                grid=(bs,),
                in_specs=specs(off),
                out_specs=out_spec,
                out_shape=out_shape,
            )(g3, c1, c2, vaf3, emb_vc, emb_func, W, b2)
        else:
            out = pl.pallas_call(
                _tc_body_acc,
                grid=(bs,),
                in_specs=[pl.BlockSpec(memory_space=pl.ANY)] + specs(off),
                out_specs=out_spec,
                out_shape=out_shape,
                input_output_aliases={0: 0},
            )(out, g3, c1, c2, vaf3, emb_vc, emb_func, W, b2)

    return out


# speedup vs baseline: 1.1067x; 1.1067x over previous
import functools

import jax
import jax.numpy as jnp
from jax import lax
from jax.experimental import pallas as pl
from jax.experimental.pallas import tpu as pltpu
from jax.experimental.pallas import tpu_sc as plsc

_NC = 2
_NS = 16
_NW = _NC * _NS
_CHUNK = 120
_NBUF = 3
_NSLICE = 4

_D_VAR = 128
_OUT = 256


def _sc_gather(table, idx_full, n_rows, slice_base):
    per_w = n_rows // _NW
    k_chunks = per_w // _CHUNK
    assert per_w % _CHUNK == 0 and k_chunks % _NBUF == 0
    mesh = plsc.VectorSubcoreMesh(core_axis_name="c", subcore_axis_name="s")

    dt = table.dtype

    @functools.partial(
        pl.kernel,
        mesh=mesh,
        out_type=jax.ShapeDtypeStruct((n_rows, _D_VAR), dt),
        scratch_types=[
            pltpu.VMEM((per_w,), jnp.int32),
        ] + [pltpu.VMEM((_CHUNK, _D_VAR), dt)] * _NBUF
          + [pltpu.SemaphoreType.DMA] * (2 * _NBUF),
    )
    def gather_kernel(table_hbm, idx_hbm, out_hbm, idxall, r0, r1, r2,
                      g0, g1, g2, w0, w1, w2):
        rows = [r0, r1, r2]
        gsem = [g0, g1, g2]
        wsem = [w0, w1, w2]
        wid = lax.axis_index("s") * _NC + lax.axis_index("c")
        base = wid * per_w

        pltpu.sync_copy(idx_hbm.at[pl.ds(slice_base + base, per_w)], idxall)

        def fire_gather(j, b):
            pltpu.async_copy(
                table_hbm.at[idxall.at[pl.ds(j * _CHUNK, _CHUNK)]],
                rows[b], gsem[b])

        def fire_wb(j, b):
            pltpu.async_copy(
                rows[b], out_hbm.at[pl.ds(base + j * _CHUNK, _CHUNK)],
                wsem[b])

        def wait_g(b):
            pltpu.make_async_copy(
                table_hbm.at[idxall.at[pl.ds(0, _CHUNK)]], rows[b],
                gsem[b]).wait()

        def wait_w(b):
            pltpu.make_async_copy(
                rows[b], out_hbm.at[pl.ds(base, _CHUNK)], wsem[b]).wait()

        def body(g, carry):
            for b in range(_NBUF):
                jb = _NBUF * g + b
                pb = (b + _NBUF - 1) % _NBUF

                @pl.when(g > 0)
                def _():
                    wait_w(b)

                fire_gather(jb, b)

                if b == 0:
                    @pl.when(g > 0)
                    def _():
                        wait_g(pb)
                        fire_wb(_NBUF * g - 1, pb)
                else:
                    wait_g(pb)
                    fire_wb(jb - 1, pb)
            return carry

        lax.fori_loop(0, k_chunks // _NBUF, body, 0)
        wait_g(_NBUF - 1)
        fire_wb(k_chunks - 1, _NBUF - 1)
        for b in range(_NBUF):
            wait_w(b)

    return gather_kernel(table, idx_full)


def _dot_t(lhs, rhs):
    return lax.dot_general(lhs, rhs, (((0,), (0,)), ((), ())),
                           preferred_element_type=jnp.float32)


def _tc_compute(g_ref, c1_ref, c2_ref, vaf_ref, vc_ref, f_ref, w_ref, b_ref,
                o_ref):
    L = o_ref.shape[1]
    g = g_ref[0, 0:L, :].astype(jnp.bfloat16)
    acc = jnp.dot(g, w_ref[0:128, :].astype(jnp.bfloat16),
                  preferred_element_type=jnp.float32)

    c1 = c1_ref[0]
    c2 = c2_ref[0]

    vc_iota = lax.broadcasted_iota(jnp.int32, (33, L), 0)
    oh_vc = ((c1 & 63) == vc_iota).astype(jnp.float32)
    wvc = jnp.dot(vc_ref[...], w_ref[128:160, :],
                  preferred_element_type=jnp.float32)
    acc += _dot_t(oh_vc, wvc)

    f_iota = lax.broadcasted_iota(jnp.int32, (65, L), 0)
    counts = (((c1 >> 6) & 127) == f_iota).astype(jnp.float32)
    counts += (((c1 >> 13) & 127) == f_iota).astype(jnp.float32)
    counts += (((c1 >> 20) & 127) == f_iota).astype(jnp.float32)
    counts += ((c2 & 127) == f_iota).astype(jnp.float32)
    counts += (((c2 >> 7) & 127) == f_iota).astype(jnp.float32)
    counts += (((c2 >> 14) & 127) == f_iota).astype(jnp.float32)
    wf = jnp.dot(f_ref[...], w_ref[160:192, :],
                 preferred_element_type=jnp.float32) * (1.0 / 6.0)
    acc += _dot_t(counts, wf)

    acc += _dot_t(vaf_ref[0], w_ref[192:193, :])
    acc += b_ref[...]

    o_ref[0] = jnp.where(acc > 0.0, acc, jnp.exp(acc) - 1.0)


def _tc_body_first(g_ref, c1_ref, c2_ref, vaf_ref, vc_ref, f_ref, w_ref,
                   b_ref, o_ref):
    _tc_compute(g_ref, c1_ref, c2_ref, vaf_ref, vc_ref, f_ref, w_ref, b_ref,
                o_ref)


def _tc_body_acc(o_prev_ref, g_ref, c1_ref, c2_ref, vaf_ref, vc_ref, f_ref,
                 w_ref, b_ref, o_ref):
    del o_prev_ref
    _tc_compute(g_ref, c1_ref, c2_ref, vaf_ref, vc_ref, f_ref, w_ref, b_ref,
                o_ref)


def kernel(x_omic, emb_var, emb_vc, emb_func, W, b):
    B, L, _ = x_omic.shape
    step = 1440 if L <= 1440 else -(-L // 1440) * 1440
    l_pad = step
    n_pad = B * l_pad
    bs = B // _NSLICE
    n_sl = bs * l_pad

    var_ids = x_omic[..., 0].astype(jnp.int32)
    var_ids = jnp.pad(var_ids, ((0, 0), (0, l_pad - L)))
    idx_flat = var_ids.reshape(n_pad)

    ids = x_omic[..., 1:8].astype(jnp.int32)
    c1 = (ids[..., 0] | (ids[..., 1] << 6) | (ids[..., 2] << 13)
          | (ids[..., 3] << 20)).reshape(B, 1, L)
    c2 = (ids[..., 4] | (ids[..., 5] << 7)
          | (ids[..., 6] << 14)).reshape(B, 1, L)
    vaf3 = x_omic[..., 8].reshape(B, 1, L)

    b2 = b.reshape(1, _OUT)
    out_shape = jax.ShapeDtypeStruct((B, L, _OUT), jnp.float32)

    def specs(off):
        return [
            pl.BlockSpec((1, l_pad, _D_VAR), lambda i: (i, 0, 0)),
            pl.BlockSpec((1, 1, L), lambda i, o=off: (i + o, 0, 0)),
            pl.BlockSpec((1, 1, L), lambda i, o=off: (i + o, 0, 0)),
            pl.BlockSpec((1, 1, L), lambda i, o=off: (i + o, 0, 0)),
            pl.BlockSpec((33, 32), lambda i: (0, 0)),
            pl.BlockSpec((65, 32), lambda i: (0, 0)),
            pl.BlockSpec((193, _OUT), lambda i: (0, 0)),
            pl.BlockSpec((1, _OUT), lambda i: (0, 0)),
        ]

    out = None
    for s in range(_NSLICE):
        gath = _sc_gather(emb_var, idx_flat, n_sl, s * n_sl)
        g3 = gath.reshape(bs, l_pad, _D_VAR)
        off = s * bs
        out_spec = pl.BlockSpec((1, L, _OUT), lambda i, o=off: (i + o, 0, 0))
        if s == 0:
            out = pl.pallas_call(
                _tc_body_first,
                grid=(bs,),
                in_specs=specs(off),
                out_specs=out_spec,
                out_shape=out_shape,
            )(g3, c1, c2, vaf3, emb_vc, emb_func, W, b2)
        else:
            out = pl.pallas_call(
                _tc_body_acc,
                grid=(bs,),
                in_specs=[pl.BlockSpec(memory_space=pl.ANY)] + specs(off),
                out_specs=out_spec,
                out_shape=out_shape,
                input_output_aliases={0: 0},
            )(out, g3, c1, c2, vaf3, emb_vc, emb_func, W, b2)

    return out


# speedup vs baseline: 1.1282x; 1.0194x over previous
import functools

import jax
import jax.numpy as jnp
from jax import lax
from jax.experimental import pallas as pl
from jax.experimental.pallas import tpu as pltpu
from jax.experimental.pallas import tpu_sc as plsc

_NC = 2
_NS = 16
_NW = _NC * _NS
_CHUNK = 120
_NBUF = 3
_NSLICE = 4

_D_VAR = 128
_OUT = 256


def _sc_gather(table, idx_full, n_rows, slice_base):
    per_w = n_rows // _NW
    k_chunks = per_w // _CHUNK
    assert per_w % _CHUNK == 0 and k_chunks % _NBUF == 0
    mesh = plsc.VectorSubcoreMesh(core_axis_name="c", subcore_axis_name="s")

    dt = table.dtype

    @functools.partial(
        pl.kernel,
        mesh=mesh,
        out_type=jax.ShapeDtypeStruct((n_rows, _D_VAR), dt),
        scratch_types=[
            pltpu.VMEM((per_w,), jnp.int32),
        ] + [pltpu.VMEM((_CHUNK, _D_VAR), dt)] * _NBUF
          + [pltpu.SemaphoreType.DMA] * (2 * _NBUF),
    )
    def gather_kernel(table_hbm, idx_hbm, out_hbm, idxall, r0, r1, r2,
                      g0, g1, g2, w0, w1, w2):
        rows = [r0, r1, r2]
        gsem = [g0, g1, g2]
        wsem = [w0, w1, w2]
        wid = lax.axis_index("s") * _NC + lax.axis_index("c")
        base = wid * per_w

        pltpu.sync_copy(idx_hbm.at[pl.ds(slice_base + base, per_w)], idxall)

        def fire_gather(j, b):
            pltpu.async_copy(
                table_hbm.at[idxall.at[pl.ds(j * _CHUNK, _CHUNK)]],
                rows[b], gsem[b])

        def fire_wb(j, b):
            pltpu.async_copy(
                rows[b], out_hbm.at[pl.ds(base + j * _CHUNK, _CHUNK)],
                wsem[b])

        def wait_g(b):
            pltpu.make_async_copy(
                table_hbm.at[idxall.at[pl.ds(0, _CHUNK)]], rows[b],
                gsem[b]).wait()

        def wait_w(b):
            pltpu.make_async_copy(
                rows[b], out_hbm.at[pl.ds(base, _CHUNK)], wsem[b]).wait()

        def body(g, carry):
            for b in range(_NBUF):
                jb = _NBUF * g + b
                pb = (b + _NBUF - 1) % _NBUF

                @pl.when(g > 0)
                def _():
                    wait_w(b)

                fire_gather(jb, b)

                if b == 0:
                    @pl.when(g > 0)
                    def _():
                        wait_g(pb)
                        fire_wb(_NBUF * g - 1, pb)
                else:
                    wait_g(pb)
                    fire_wb(jb - 1, pb)
            return carry

        lax.fori_loop(0, k_chunks // _NBUF, body, 0)
        wait_g(_NBUF - 1)
        fire_wb(k_chunks - 1, _NBUF - 1)
        for b in range(_NBUF):
            wait_w(b)

    return gather_kernel(table, idx_full)


def _dot_t(lhs, rhs):
    return lax.dot_general(lhs, rhs, (((0,), (0,)), ((), ())),
                           preferred_element_type=jnp.float32)


def _tc_compute(g_ref, c1_ref, c2_ref, vaf_ref, vc_ref, f_ref, w_ref, b_ref,
                o_ref):
    L = o_ref.shape[1]
    g = g_ref[0, 0:L, :].astype(jnp.bfloat16)
    acc = jnp.dot(g, w_ref[0:128, :].astype(jnp.bfloat16),
                  preferred_element_type=jnp.float32)

    c1 = c1_ref[0]
    c2 = c2_ref[0]

    vc_iota = lax.broadcasted_iota(jnp.int32, (33, L), 0)
    oh_vc = ((c1 & 63) == vc_iota).astype(jnp.bfloat16)
    wvc = jnp.dot(vc_ref[...], w_ref[128:160, :],
                  preferred_element_type=jnp.float32)
    acc += _dot_t(oh_vc, wvc.astype(jnp.bfloat16))

    f_iota = lax.broadcasted_iota(jnp.int32, (65, L), 0)
    counts = (((c1 >> 6) & 127) == f_iota).astype(jnp.bfloat16)
    counts += (((c1 >> 13) & 127) == f_iota).astype(jnp.bfloat16)
    counts += (((c1 >> 20) & 127) == f_iota).astype(jnp.bfloat16)
    counts += ((c2 & 127) == f_iota).astype(jnp.bfloat16)
    counts += (((c2 >> 7) & 127) == f_iota).astype(jnp.bfloat16)
    counts += (((c2 >> 14) & 127) == f_iota).astype(jnp.bfloat16)
    wf = jnp.dot(f_ref[...], w_ref[160:192, :],
                 preferred_element_type=jnp.float32) * (1.0 / 6.0)
    acc += _dot_t(counts, wf.astype(jnp.bfloat16))

    acc += _dot_t(vaf_ref[0], w_ref[192:193, :])
    acc += b_ref[...]

    o_ref[0] = jnp.where(acc > 0.0, acc, jnp.exp(acc) - 1.0)


def _tc_body_first(g_ref, c1_ref, c2_ref, vaf_ref, vc_ref, f_ref, w_ref,
                   b_ref, o_ref):
    _tc_compute(g_ref, c1_ref, c2_ref, vaf_ref, vc_ref, f_ref, w_ref, b_ref,
                o_ref)


def _tc_body_acc(o_prev_ref, g_ref, c1_ref, c2_ref, vaf_ref, vc_ref, f_ref,
                 w_ref, b_ref, o_ref):
    del o_prev_ref
    _tc_compute(g_ref, c1_ref, c2_ref, vaf_ref, vc_ref, f_ref, w_ref, b_ref,
                o_ref)


def kernel(x_omic, emb_var, emb_vc, emb_func, W, b):
    B, L, _ = x_omic.shape
    step = 1440 if L <= 1440 else -(-L // 1440) * 1440
    l_pad = step
    n_pad = B * l_pad
    bs = B // _NSLICE
    n_sl = bs * l_pad

    var_ids = x_omic[..., 0].astype(jnp.int32)
    var_ids = jnp.pad(var_ids, ((0, 0), (0, l_pad - L)))
    idx_flat = var_ids.reshape(n_pad)

    ids = x_omic[..., 1:8].astype(jnp.int32)
    c1 = (ids[..., 0] | (ids[..., 1] << 6) | (ids[..., 2] << 13)
          | (ids[..., 3] << 20)).reshape(B, 1, L)
    c2 = (ids[..., 4] | (ids[..., 5] << 7)
          | (ids[..., 6] << 14)).reshape(B, 1, L)
    vaf3 = x_omic[..., 8].reshape(B, 1, L)

    b2 = b.reshape(1, _OUT)
    out_shape = jax.ShapeDtypeStruct((B, L, _OUT), jnp.float32)

    def specs(off):
        return [
            pl.BlockSpec((1, l_pad, _D_VAR), lambda i: (i, 0, 0)),
            pl.BlockSpec((1, 1, L), lambda i, o=off: (i + o, 0, 0)),
            pl.BlockSpec((1, 1, L), lambda i, o=off: (i + o, 0, 0)),
            pl.BlockSpec((1, 1, L), lambda i, o=off: (i + o, 0, 0)),
            pl.BlockSpec((33, 32), lambda i: (0, 0)),
            pl.BlockSpec((65, 32), lambda i: (0, 0)),
            pl.BlockSpec((193, _OUT), lambda i: (0, 0)),
            pl.BlockSpec((1, _OUT), lambda i: (0, 0)),
        ]

    out = None
    for s in range(_NSLICE):
        gath = _sc_gather(emb_var, idx_flat, n_sl, s * n_sl)
        g3 = gath.reshape(bs, l_pad, _D_VAR)
        off = s * bs
        out_spec = pl.BlockSpec((1, L, _OUT), lambda i, o=off: (i + o, 0, 0))
        if s == 0:
            out = pl.pallas_call(
                _tc_body_first,
                grid=(bs,),
                in_specs=specs(off),
                out_specs=out_spec,
                out_shape=out_shape,
            )(g3, c1, c2, vaf3, emb_vc, emb_func, W, b2)
        else:
            out = pl.pallas_call(
                _tc_body_acc,
                grid=(bs,),
                in_specs=[pl.BlockSpec(memory_space=pl.ANY)] + specs(off),
                out_specs=out_spec,
                out_shape=out_shape,
                input_output_aliases={0: 0},
            )(out, g3, c1, c2, vaf3, emb_vc, emb_func, W, b2)

    return out


# speedup vs baseline: 1.1439x; 1.0139x over previous
import functools

import jax
import jax.numpy as jnp
from jax import lax
from jax.experimental import pallas as pl
from jax.experimental.pallas import tpu as pltpu
from jax.experimental.pallas import tpu_sc as plsc

_NC = 2
_NS = 16
_NW = _NC * _NS
_CHUNK = 120
_NBUF = 3
_NSLICE = 4

_D_VAR = 128
_OUT = 256


def _sc_gather(table, idx_full, n_rows, slice_base):
    per_w = n_rows // _NW
    k_chunks = per_w // _CHUNK
    assert per_w % _CHUNK == 0 and k_chunks % _NBUF == 0
    mesh = plsc.VectorSubcoreMesh(core_axis_name="c", subcore_axis_name="s")

    dt = table.dtype

    @functools.partial(
        pl.kernel,
        mesh=mesh,
        out_type=jax.ShapeDtypeStruct((n_rows, _D_VAR), dt),
        scratch_types=[
            pltpu.VMEM((per_w,), jnp.int32),
        ] + [pltpu.VMEM((_CHUNK, _D_VAR), dt)] * _NBUF
          + [pltpu.SemaphoreType.DMA] * (2 * _NBUF),
    )
    def gather_kernel(table_hbm, idx_hbm, out_hbm, idxall, r0, r1, r2,
                      g0, g1, g2, w0, w1, w2):
        rows = [r0, r1, r2]
        gsem = [g0, g1, g2]
        wsem = [w0, w1, w2]
        wid = lax.axis_index("s") * _NC + lax.axis_index("c")
        base = wid * per_w

        pltpu.sync_copy(idx_hbm.at[pl.ds(slice_base + base, per_w)], idxall)

        def fire_gather(j, b):
            pltpu.async_copy(
                table_hbm.at[idxall.at[pl.ds(j * _CHUNK, _CHUNK)]],
                rows[b], gsem[b])

        def fire_wb(j, b):
            pltpu.async_copy(
                rows[b], out_hbm.at[pl.ds(base + j * _CHUNK, _CHUNK)],
                wsem[b])

        def wait_g(b):
            pltpu.make_async_copy(
                table_hbm.at[idxall.at[pl.ds(0, _CHUNK)]], rows[b],
                gsem[b]).wait()

        def wait_w(b):
            pltpu.make_async_copy(
                rows[b], out_hbm.at[pl.ds(base, _CHUNK)], wsem[b]).wait()

        def body(g, carry):
            for b in range(_NBUF):
                jb = _NBUF * g + b
                pb = (b + _NBUF - 1) % _NBUF

                @pl.when(g > 0)
                def _():
                    wait_w(b)

                fire_gather(jb, b)

                if b == 0:
                    @pl.when(g > 0)
                    def _():
                        wait_g(pb)
                        fire_wb(_NBUF * g - 1, pb)
                else:
                    wait_g(pb)
                    fire_wb(jb - 1, pb)
            return carry

        lax.fori_loop(0, k_chunks // _NBUF, body, 0)
        wait_g(_NBUF - 1)
        fire_wb(k_chunks - 1, _NBUF - 1)
        for b in range(_NBUF):
            wait_w(b)

    return gather_kernel(table, idx_full)


def _dot_t(lhs, rhs):
    return lax.dot_general(lhs, rhs, (((0,), (0,)), ((), ())),
                           preferred_element_type=jnp.float32)


def _tc_compute(g_ref, c1_ref, c2_ref, vaf_ref, vc_ref, f_ref, w_ref, b_ref,
                o_ref):
    L = o_ref.shape[1]
    g = g_ref[0, 0:L, :].astype(jnp.bfloat16)
    acc = jnp.dot(g, w_ref[0:128, :].astype(jnp.bfloat16),
                  preferred_element_type=jnp.float32)

    c1 = c1_ref[0]
    c2 = c2_ref[0]

    vc_iota = lax.broadcasted_iota(jnp.int32, (40, L), 0)
    oh_vc = ((c1 & 63) == vc_iota).astype(jnp.bfloat16)
    f_iota = lax.broadcasted_iota(jnp.int32, (72, L), 0)
    counts = (((c1 >> 6) & 127) == f_iota).astype(jnp.bfloat16)
    counts += (((c1 >> 13) & 127) == f_iota).astype(jnp.bfloat16)
    counts += (((c1 >> 20) & 127) == f_iota).astype(jnp.bfloat16)
    counts += ((c2 & 127) == f_iota).astype(jnp.bfloat16)
    counts += (((c2 >> 7) & 127) == f_iota).astype(jnp.bfloat16)
    counts += (((c2 >> 14) & 127) == f_iota).astype(jnp.bfloat16)
    lhs = jnp.concatenate([oh_vc, counts], axis=0)

    wvc = jnp.dot(vc_ref[...], w_ref[128:160, :],
                  preferred_element_type=jnp.float32)
    wf = jnp.dot(f_ref[...], w_ref[160:192, :],
                 preferred_element_type=jnp.float32) * (1.0 / 6.0)
    zpad = jnp.zeros((7, _OUT), jnp.float32)
    rhs = jnp.concatenate([wvc, zpad, wf, zpad], axis=0)
    acc += _dot_t(lhs, rhs.astype(jnp.bfloat16))

    acc += _dot_t(vaf_ref[0], w_ref[192:193, :])
    acc += b_ref[...]

    o_ref[0] = jnp.where(acc > 0.0, acc, jnp.exp(acc) - 1.0)


def _tc_body_first(g_ref, c1_ref, c2_ref, vaf_ref, vc_ref, f_ref, w_ref,
                   b_ref, o_ref):
    _tc_compute(g_ref, c1_ref, c2_ref, vaf_ref, vc_ref, f_ref, w_ref, b_ref,
                o_ref)


def _tc_body_acc(o_prev_ref, g_ref, c1_ref, c2_ref, vaf_ref, vc_ref, f_ref,
                 w_ref, b_ref, o_ref):
    del o_prev_ref
    _tc_compute(g_ref, c1_ref, c2_ref, vaf_ref, vc_ref, f_ref, w_ref, b_ref,
                o_ref)


def kernel(x_omic, emb_var, emb_vc, emb_func, W, b):
    B, L, _ = x_omic.shape
    step = 1440 if L <= 1440 else -(-L // 1440) * 1440
    l_pad = step
    n_pad = B * l_pad
    bs = B // _NSLICE
    n_sl = bs * l_pad

    var_ids = x_omic[..., 0].astype(jnp.int32)
    var_ids = jnp.pad(var_ids, ((0, 0), (0, l_pad - L)))
    idx_flat = var_ids.reshape(n_pad)

    ids = x_omic[..., 1:8].astype(jnp.int32)
    c1 = (ids[..., 0] | (ids[..., 1] << 6) | (ids[..., 2] << 13)
          | (ids[..., 3] << 20)).reshape(B, 1, L)
    c2 = (ids[..., 4] | (ids[..., 5] << 7)
          | (ids[..., 6] << 14)).reshape(B, 1, L)
    vaf3 = x_omic[..., 8].reshape(B, 1, L)

    b2 = b.reshape(1, _OUT)
    out_shape = jax.ShapeDtypeStruct((B, L, _OUT), jnp.float32)

    def specs(off):
        return [
            pl.BlockSpec((1, l_pad, _D_VAR), lambda i: (i, 0, 0)),
            pl.BlockSpec((1, 1, L), lambda i, o=off: (i + o, 0, 0)),
            pl.BlockSpec((1, 1, L), lambda i, o=off: (i + o, 0, 0)),
            pl.BlockSpec((1, 1, L), lambda i, o=off: (i + o, 0, 0)),
            pl.BlockSpec((33, 32), lambda i: (0, 0)),
            pl.BlockSpec((65, 32), lambda i: (0, 0)),
            pl.BlockSpec((193, _OUT), lambda i: (0, 0)),
            pl.BlockSpec((1, _OUT), lambda i: (0, 0)),
        ]

    out = None
    for s in range(_NSLICE):
        gath = _sc_gather(emb_var, idx_flat, n_sl, s * n_sl)
        g3 = gath.reshape(bs, l_pad, _D_VAR)
        off = s * bs
        out_spec = pl.BlockSpec((1, L, _OUT), lambda i, o=off: (i + o, 0, 0))
        if s == 0:
            out = pl.pallas_call(
                _tc_body_first,
                grid=(bs,),
                in_specs=specs(off),
                out_specs=out_spec,
                out_shape=out_shape,
            )(g3, c1, c2, vaf3, emb_vc, emb_func, W, b2)
        else:
            out = pl.pallas_call(
                _tc_body_acc,
                grid=(bs,),
                in_specs=[pl.BlockSpec(memory_space=pl.ANY)] + specs(off),
                out_specs=out_spec,
                out_shape=out_shape,
                input_output_aliases={0: 0},
            )(out, g3, c1, c2, vaf3, emb_vc, emb_func, W, b2)

    return out
